# baseline (speedup 1.0000x reference)
import functools

import jax
import jax.numpy as jnp
from jax import lax
from jax.experimental import pallas as pl
from jax.experimental.pallas import tpu as pltpu
from jax.experimental.pallas import tpu_sc as plsc

_NC = 2
_NS = 16
_NW = _NC * _NS


def _sc_gather_cols(embT, idx):
    D, V = embT.shape
    B = idx.shape[0]
    b_per_w = B // _NW
    mesh = plsc.VectorSubcoreMesh(core_axis_name="c", subcore_axis_name="s")

    @functools.partial(
        pl.kernel,
        mesh=mesh,
        compiler_params=pltpu.CompilerParams(needs_layout_passes=False),
        out_type=jax.ShapeDtypeStruct((D, B), jnp.float32),
        scratch_types=[
            pltpu.VMEM((b_per_w,), jnp.int32),
            pltpu.VMEM((D, 32 * 128), jnp.float32),
            pltpu.VMEM((D, 32 * 128), jnp.float32),
            pltpu.VMEM((D, b_per_w), jnp.float32),
            pltpu.SemaphoreType.DMA,
            pltpu.SemaphoreType.DMA,
        ],
    )
    def gather_kernel(tab_hbm, idx_hbm, out_hbm, idx_v, win0, win1, cols_v,
                      sem0, sem1):
        wid = lax.axis_index("s") * _NC + lax.axis_index("c")
        base = wid * b_per_w
        pltpu.sync_copy(idx_hbm.at[pl.ds(base, b_per_w)], idx_v)

        iota16 = lax.iota(jnp.int32, 16)
        R = 32
        n_rounds = b_per_w // R

        def fire(r, win, sem):
            for g in range(R // 16):
                vec = idx_v[pl.ds(r * R + g * 16, 16)]
                alv = (vec >> 7) << 7
                for j in range(16):
                    k = g * 16 + j
                    al = pl.multiple_of(alv[j], 128)
                    pltpu.make_async_copy(
                        tab_hbm.at[:, pl.ds(al, 128)],
                        win.at[:, pl.ds(k * 128, 128)],
                        sem,
                    ).start()

        def drain_extract(r, win, sem):
            @pl.loop(0, R)
            def _(k):
                pltpu.make_async_copy(
                    tab_hbm.at[:, pl.ds(0, 128)],
                    win.at[:, pl.ds(0, 128)],
                    sem,
                ).wait()

            for g in range(R // 16):
                vec = idx_v[pl.ds(r * R + g * 16, 16)]
                pos = g * (16 * 128) + iota16 * 128 + (vec & 127)
                for c in range(D):
                    row = jnp.full((16,), c, dtype=jnp.int32)
                    vals = plsc.load_gather(win, [row, pos])
                    cols_v[c, pl.ds(r * R + g * 16, 16)] = vals

        fire(0, win0, sem0)

        @pl.loop(0, n_rounds // 2)
        def _(p):
            fire(2 * p + 1, win1, sem1)
            drain_extract(2 * p, win0, sem0)

            @pl.when(p < n_rounds // 2 - 1)
            def _():
                fire(2 * p + 2, win0, sem0)

            drain_extract(2 * p + 1, win1, sem1)

        pltpu.sync_copy(cols_v, out_hbm.at[:, pl.ds(base, b_per_w)])

    return gather_kernel(embT, idx)


def _mlp_kernel(xT_ref, w1_ref, b1_ref, w2_ref, b2_ref, w3_ref, b3_ref,
                wo_ref, bo_ref, o_ref):
    h = jnp.dot(w1_ref[...], xT_ref[...], preferred_element_type=jnp.float32)
    h = jnp.maximum(h + b1_ref[...], 0.0)
    h = jnp.dot(w2_ref[...], h, preferred_element_type=jnp.float32)
    h = jnp.maximum(h + b2_ref[...], 0.0)
    h = jnp.dot(w3_ref[...], h, preferred_element_type=jnp.float32)
    h = jnp.maximum(h + b3_ref[...], 0.0)
    o = jnp.dot(wo_ref[...], h, preferred_element_type=jnp.float32)
    o_ref[...] = o + bo_ref[...]


def _tc_mlp(xT, W1, b1, W2, b2, W3, b3, Wo, bo, blk):
    D, B = xT.shape
    H = W2.shape[0]
    O = Wo.shape[0]
    full = lambda shape: pl.BlockSpec(shape, lambda i: (0, 0))
    return pl.pallas_call(
        _mlp_kernel,
        grid=(B // blk,),
        in_specs=[
            pl.BlockSpec((D, blk), lambda i: (0, i)),
            full((H, D)), full((H, 1)),
            full((H, H)), full((H, 1)),
            full((H, H)), full((H, 1)),
            full((O, H)), full((O, 1)),
        ],
        out_specs=pl.BlockSpec((O, blk), lambda i: (0, i)),
        out_shape=jax.ShapeDtypeStruct((O, B), jnp.float32),
    )(xT, W1, b1, W2, b2, W3, b3, Wo, bo)


def kernel(x, emb, W1, b1, W2, b2, W3, b3, Wm, bm, Wc, bc, Ws, bs):
    B = x.shape[0]
    idx = x[:, 0].astype(jnp.int32)
    embT = emb.T
    Wo = jnp.concatenate([Wm, Wc, Ws], axis=0)
    bo = jnp.concatenate([bm, bc, bs], axis=0)[:, None]
    h = B // 2
    xT0 = _sc_gather_cols(embT, idx[:h])
    xT1 = _sc_gather_cols(embT, idx[h:])
    mlp = lambda xT: _tc_mlp(xT, W1, b1[:, None], W2, b2[:, None],
                             W3, b3[:, None], Wo, bo, blk=2048)
    return jnp.concatenate([mlp(xT0), mlp(xT1)], axis=1).T


# speedup vs baseline: 1.0917x; 1.0917x over previous
import functools

import jax
import jax.numpy as jnp
from jax import lax
from jax.experimental import pallas as pl
from jax.experimental.pallas import tpu as pltpu
from jax.experimental.pallas import tpu_sc as plsc

_NC = 2
_NS = 16
_NW = _NC * _NS


def _sc_gather_cols(embT, xT):
    D, V = embT.shape
    B = xT.shape[1]
    b_per_w = B // _NW
    mesh = plsc.VectorSubcoreMesh(core_axis_name="c", subcore_axis_name="s")

    @functools.partial(
        pl.kernel,
        mesh=mesh,
        compiler_params=pltpu.CompilerParams(needs_layout_passes=False),
        out_type=jax.ShapeDtypeStruct((D, B), jnp.float32),
        scratch_types=[
            pltpu.VMEM((1, b_per_w), jnp.float32),
            pltpu.VMEM((D, 32 * 128), jnp.float32),
            pltpu.VMEM((D, 32 * 128), jnp.float32),
            pltpu.VMEM((D, b_per_w), jnp.float32),
            pltpu.SemaphoreType.DMA,
            pltpu.SemaphoreType.DMA,
        ],
    )
    def gather_kernel(tab_hbm, x_hbm, out_hbm, xf_v, win0, win1, cols_v,
                      sem0, sem1):
        wid = lax.axis_index("s") * _NC + lax.axis_index("c")
        base = pl.multiple_of(wid * b_per_w, 128)
        pltpu.sync_copy(x_hbm.at[:, pl.ds(base, b_per_w)], xf_v)

        def idx16(off):
            return xf_v[0, pl.ds(off, 16)].astype(jnp.int32)

        iota16 = lax.iota(jnp.int32, 16)
        R = 32
        n_rounds = b_per_w // R

        def fire(r, win, sem):
            for g in range(R // 16):
                vec = idx16(r * R + g * 16)
                alv = (vec >> 7) << 7
                for j in range(16):
                    k = g * 16 + j
                    al = pl.multiple_of(alv[j], 128)
                    pltpu.make_async_copy(
                        tab_hbm.at[:, pl.ds(al, 128)],
                        win.at[:, pl.ds(k * 128, 128)],
                        sem,
                    ).start()

        def drain_extract(r, win, sem):
            @pl.loop(0, R)
            def _(k):
                pltpu.make_async_copy(
                    tab_hbm.at[:, pl.ds(0, 128)],
                    win.at[:, pl.ds(0, 128)],
                    sem,
                ).wait()

            for g in range(R // 16):
                vec = idx16(r * R + g * 16)
                pos = g * (16 * 128) + iota16 * 128 + (vec & 127)
                for c in range(D):
                    row = jnp.full((16,), c, dtype=jnp.int32)
                    vals = plsc.load_gather(win, [row, pos])
                    cols_v[c, pl.ds(r * R + g * 16, 16)] = vals

        fire(0, win0, sem0)

        @pl.loop(0, n_rounds // 2)
        def _(p):
            fire(2 * p + 1, win1, sem1)
            drain_extract(2 * p, win0, sem0)

            @pl.when(p < n_rounds // 2 - 1)
            def _():
                fire(2 * p + 2, win0, sem0)

            drain_extract(2 * p + 1, win1, sem1)

        pltpu.sync_copy(cols_v, out_hbm.at[:, pl.ds(base, b_per_w)])

    return gather_kernel(embT, xT)


def _mlp_kernel(xT_ref, w1_ref, b1_ref, w2_ref, b2_ref, w3_ref, b3_ref,
                wo_ref, bo_ref, o_ref):
    h = jnp.dot(w1_ref[...], xT_ref[...], preferred_element_type=jnp.float32)
    h = jnp.maximum(h + b1_ref[...], 0.0)
    h = jnp.dot(w2_ref[...], h, preferred_element_type=jnp.float32)
    h = jnp.maximum(h + b2_ref[...], 0.0)
    h = jnp.dot(w3_ref[...], h, preferred_element_type=jnp.float32)
    h = jnp.maximum(h + b3_ref[...], 0.0)
    o = jnp.dot(wo_ref[...], h, preferred_element_type=jnp.float32)
    o_ref[...] = o + bo_ref[...]


def _tc_mlp(xT, W1, b1, W2, b2, W3, b3, Wo, bo, blk):
    D, B = xT.shape
    H = W2.shape[0]
    O = Wo.shape[0]
    full = lambda shape: pl.BlockSpec(shape, lambda i: (0, 0))
    return pl.pallas_call(
        _mlp_kernel,
        grid=(B // blk,),
        in_specs=[
            pl.BlockSpec((D, blk), lambda i: (0, i)),
            full((H, D)), full((H, 1)),
            full((H, H)), full((H, 1)),
            full((H, H)), full((H, 1)),
            full((O, H)), full((O, 1)),
        ],
        out_specs=pl.BlockSpec((O, blk), lambda i: (0, i)),
        out_shape=jax.ShapeDtypeStruct((O, B), jnp.float32),
    )(xT, W1, b1, W2, b2, W3, b3, Wo, bo)


def kernel(x, emb, W1, b1, W2, b2, W3, b3, Wm, bm, Wc, bc, Ws, bs):
    Wo = jnp.concatenate([Wm, Wc, Ws], axis=0)
    bo = jnp.concatenate([bm, bc, bs], axis=0)[:, None]
    xT = _sc_gather_cols(emb.T, x.T)
    oT = _tc_mlp(xT, W1, b1[:, None], W2, b2[:, None],
                 W3, b3[:, None], Wo, bo, blk=2048)
    return oT.T


# speedup vs baseline: 1.0920x; 1.0002x over previous
import functools

import jax
import jax.numpy as jnp
from jax import lax
from jax.experimental import pallas as pl
from jax.experimental.pallas import tpu as pltpu
from jax.experimental.pallas import tpu_sc as plsc

_NC = 2
_NS = 16
_NW = _NC * _NS


def _sc_gather_cols(embT, xT):
    D, V = embT.shape
    B = xT.shape[1]
    b_per_w = B // _NW
    mesh = plsc.VectorSubcoreMesh(core_axis_name="c", subcore_axis_name="s")

    @functools.partial(
        pl.kernel,
        mesh=mesh,
        compiler_params=pltpu.CompilerParams(needs_layout_passes=False),
        out_type=jax.ShapeDtypeStruct((D, B), jnp.float32),
        scratch_types=[
            pltpu.VMEM((1, b_per_w), jnp.float32),
            pltpu.VMEM((D, 32 * 128), jnp.float32),
            pltpu.VMEM((D, 32 * 128), jnp.float32),
            pltpu.VMEM((D, b_per_w), jnp.float32),
            pltpu.SemaphoreType.DMA,
            pltpu.SemaphoreType.DMA,
        ],
    )
    def gather_kernel(tab_hbm, x_hbm, out_hbm, xf_v, win0, win1, cols_v,
                      sem0, sem1):
        wid = lax.axis_index("s") * _NC + lax.axis_index("c")
        base = pl.multiple_of(wid * b_per_w, 128)
        pltpu.sync_copy(x_hbm.at[:, pl.ds(base, b_per_w)], xf_v)

        def idx16(off):
            return xf_v[0, pl.ds(off, 16)].astype(jnp.int32)

        iota16 = lax.iota(jnp.int32, 16)
        R = 32
        n_rounds = b_per_w // R

        def fire(r, win, sem):
            for g in range(R // 16):
                vec = idx16(r * R + g * 16)
                alv = (vec >> 7) << 7
                for j in range(16):
                    k = g * 16 + j
                    al = pl.multiple_of(alv[j], 128)
                    pltpu.make_async_copy(
                        tab_hbm.at[:, pl.ds(al, 128)],
                        win.at[:, pl.ds(k * 128, 128)],
                        sem,
                    ).start()

        def drain_extract(r, win, sem):
            @pl.loop(0, R)
            def _(k):
                pltpu.make_async_copy(
                    tab_hbm.at[:, pl.ds(0, 128)],
                    win.at[:, pl.ds(0, 128)],
                    sem,
                ).wait()

            for g in range(R // 16):
                vec = idx16(r * R + g * 16)
                pos = g * (16 * 128) + iota16 * 128 + (vec & 127)
                for c in range(D):
                    row = jnp.full((16,), c, dtype=jnp.int32)
                    vals = plsc.load_gather(win, [row, pos])
                    cols_v[c, pl.ds(r * R + g * 16, 16)] = vals

        fire(0, win0, sem0)

        @pl.loop(0, n_rounds // 2)
        def _(p):
            fire(2 * p + 1, win1, sem1)
            drain_extract(2 * p, win0, sem0)

            @pl.when(p < n_rounds // 2 - 1)
            def _():
                fire(2 * p + 2, win0, sem0)

            drain_extract(2 * p + 1, win1, sem1)

        pltpu.sync_copy(cols_v, out_hbm.at[:, pl.ds(base, b_per_w)])

    return gather_kernel(embT, xT)


def _mlp_kernel(xT_ref, w1_ref, b1_ref, w2_ref, b2_ref, w3_ref, b3_ref,
                wo_ref, bo_ref, o_ref):
    bf = jnp.bfloat16
    h = jnp.dot(w1_ref[...], xT_ref[...], preferred_element_type=jnp.float32)
    h = jnp.maximum(h + b1_ref[...], 0.0).astype(bf)
    h = jnp.dot(w2_ref[...].astype(bf), h, preferred_element_type=jnp.float32)
    h = jnp.maximum(h + b2_ref[...], 0.0).astype(bf)
    h = jnp.dot(w3_ref[...].astype(bf), h, preferred_element_type=jnp.float32)
    h = jnp.maximum(h + b3_ref[...], 0.0).astype(bf)
    o = jnp.dot(wo_ref[...].astype(bf), h, preferred_element_type=jnp.float32)
    o_ref[...] = o + bo_ref[...]


def _tc_mlp(xT, W1, b1, W2, b2, W3, b3, Wo, bo, blk):
    D, B = xT.shape
    H = W2.shape[0]
    O = Wo.shape[0]
    full = lambda shape: pl.BlockSpec(shape, lambda i: (0, 0))
    return pl.pallas_call(
        _mlp_kernel,
        grid=(B // blk,),
        in_specs=[
            pl.BlockSpec((D, blk), lambda i: (0, i)),
            full((H, D)), full((H, 1)),
            full((H, H)), full((H, 1)),
            full((H, H)), full((H, 1)),
            full((O, H)), full((O, 1)),
        ],
        out_specs=pl.BlockSpec((O, blk), lambda i: (0, i)),
        out_shape=jax.ShapeDtypeStruct((O, B), jnp.float32),
    )(xT, W1, b1, W2, b2, W3, b3, Wo, bo)


def kernel(x, emb, W1, b1, W2, b2, W3, b3, Wm, bm, Wc, bc, Ws, bs):
    Wo = jnp.concatenate([Wm, Wc, Ws], axis=0)
    bo = jnp.concatenate([bm, bc, bs], axis=0)[:, None]
    xT = _sc_gather_cols(emb.T, x.T)
    oT = _tc_mlp(xT, W1, b1[:, None], W2, b2[:, None],
                 W3, b3[:, None], Wo, bo, blk=2048)
    return oT.T


# speedup vs baseline: 1.1283x; 1.0333x over previous
import functools

import jax
import jax.numpy as jnp
from jax import lax
from jax.experimental import pallas as pl
from jax.experimental.pallas import tpu as pltpu
from jax.experimental.pallas import tpu_sc as plsc

_NC = 2
_NS = 16
_NW = _NC * _NS


def _sc_gather_cols(embT, xT):
    D, V = embT.shape
    B = xT.shape[1]
    b_per_w = B // _NW
    mesh = plsc.VectorSubcoreMesh(core_axis_name="c", subcore_axis_name="s")

    @functools.partial(
        pl.kernel,
        mesh=mesh,
        compiler_params=pltpu.CompilerParams(needs_layout_passes=False),
        out_type=jax.ShapeDtypeStruct((D, B), jnp.float32),
        scratch_types=[
            pltpu.VMEM((1, b_per_w), jnp.float32),
            pltpu.VMEM((D, 32 * 128), jnp.float32),
            pltpu.VMEM((D, 32 * 128), jnp.float32),
            pltpu.VMEM((D, b_per_w), jnp.float32),
            pltpu.SemaphoreType.DMA,
            pltpu.SemaphoreType.DMA,
        ],
    )
    def gather_kernel(tab_hbm, x_hbm, out_hbm, xf_v, win0, win1, cols_v,
                      sem0, sem1):
        wid = lax.axis_index("s") * _NC + lax.axis_index("c")
        base = pl.multiple_of(wid * b_per_w, 128)
        pltpu.sync_copy(x_hbm.at[:, pl.ds(base, b_per_w)], xf_v)

        def idx16(off):
            return xf_v[0, pl.ds(off, 16)].astype(jnp.int32)

        iota16 = lax.iota(jnp.int32, 16)
        R = 32
        n_rounds = b_per_w // R

        def fire(r, win, sem):
            for g in range(R // 16):
                vec = idx16(r * R + g * 16)
                alv = (vec >> 7) << 7
                for j in range(16):
                    k = g * 16 + j
                    al = pl.multiple_of(alv[j], 128)
                    pltpu.make_async_copy(
                        tab_hbm.at[:, pl.ds(al, 128)],
                        win.at[:, pl.ds(k * 128, 128)],
                        sem,
                    ).start()

        def drain_extract(r, win, sem):
            @pl.loop(0, R)
            def _(k):
                pltpu.make_async_copy(
                    tab_hbm.at[:, pl.ds(0, 128)],
                    win.at[:, pl.ds(0, 128)],
                    sem,
                ).wait()

            for g in range(R // 16):
                vec = idx16(r * R + g * 16)
                pos = g * (16 * 128) + iota16 * 128 + (vec & 127)
                for c in range(D):
                    row = jnp.full((16,), c, dtype=jnp.int32)
                    vals = plsc.load_gather(win, [row, pos])
                    cols_v[c, pl.ds(r * R + g * 16, 16)] = vals

        fire(0, win0, sem0)

        @pl.loop(0, n_rounds // 2)
        def _(p):
            fire(2 * p + 1, win1, sem1)
            drain_extract(2 * p, win0, sem0)

            @pl.when(p < n_rounds // 2 - 1)
            def _():
                fire(2 * p + 2, win0, sem0)

            drain_extract(2 * p + 1, win1, sem1)

        pltpu.sync_copy(cols_v, out_hbm.at[:, pl.ds(base, b_per_w)])

    return gather_kernel(embT, xT)


def _mlp_kernel(xT_ref, w1_ref, b1_ref, w2_ref, b2_ref, w3_ref, b3_ref,
                wo_ref, bo_ref, o_ref):
    h = jnp.dot(w1_ref[...], xT_ref[...], preferred_element_type=jnp.float32)
    h = jnp.maximum(h + b1_ref[...], 0.0)
    h = jnp.dot(w2_ref[...], h, preferred_element_type=jnp.float32)
    h = jnp.maximum(h + b2_ref[...], 0.0)
    h = jnp.dot(w3_ref[...], h, preferred_element_type=jnp.float32)
    h = jnp.maximum(h + b3_ref[...], 0.0)
    o = jnp.dot(wo_ref[...], h, preferred_element_type=jnp.float32)
    o_ref[...] = o + bo_ref[...]


def _tc_mlp(xT, W1, b1, W2, b2, W3, b3, Wo, bo, blk):
    D, B = xT.shape
    H = W2.shape[0]
    O = Wo.shape[0]
    full = lambda shape: pl.BlockSpec(shape, lambda i: (0, 0))
    return pl.pallas_call(
        _mlp_kernel,
        grid=(B // blk,),
        in_specs=[
            pl.BlockSpec((D, blk), lambda i: (0, i)),
            full((H, D)), full((H, 1)),
            full((H, H)), full((H, 1)),
            full((H, H)), full((H, 1)),
            full((O, H)), full((O, 1)),
        ],
        out_specs=pl.BlockSpec((O, blk), lambda i: (0, i)),
        out_shape=jax.ShapeDtypeStruct((O, B), jnp.float32),
    )(xT, W1, b1, W2, b2, W3, b3, Wo, bo)


def kernel(x, emb, W1, b1, W2, b2, W3, b3, Wm, bm, Wc, bc, Ws, bs):
    Wo = jnp.concatenate([Wm, Wc, Ws], axis=0)
    bo = jnp.concatenate([bm, bc, bs], axis=0)[:, None]
    xT = _sc_gather_cols(emb.T, x.T)
    oT = _tc_mlp(xT, W1, b1[:, None], W2, b2[:, None],
                 W3, b3[:, None], Wo, bo, blk=16384)
    return oT.T
